# Initial kernel scaffold; baseline (speedup 1.0000x reference)
import jax
import jax.numpy as jnp
from jax.experimental import pallas as pl


def kernel(x, segments, token_table, pos_table, seg_table, ln_gamma, ln_beta):
    raise NotImplementedError("write your pallas kernel here")


# speedup vs baseline: 6.1716x; 6.1716x over previous
import functools

import jax
import jax.numpy as jnp
from jax import lax
from jax.experimental import pallas as pl
from jax.experimental.pallas import tpu as pltpu
from jax.experimental.pallas import tpu_sc as plsc


def _sc_gather(table, flat_idx):
    V, D = table.shape
    N = flat_idx.shape[0]
    info = plsc.get_sparse_core_info()
    NW = info.num_cores * info.num_subcores
    per_w = N // NW
    CH = 400
    n_ch = per_w // CH
    assert per_w % CH == 0 and N % NW == 0

    mesh = plsc.VectorSubcoreMesh(core_axis_name="c", subcore_axis_name="s")

    @functools.partial(
        pl.kernel,
        mesh=mesh,
        out_type=jax.ShapeDtypeStruct((N, D), jnp.float32),
        scratch_types=[
            pltpu.VMEM((CH,), jnp.int32),
            pltpu.VMEM((CH, D), jnp.float32),
            pltpu.SemaphoreType.DMA,
        ],
    )
    def k(table_hbm, idx_hbm, out_hbm, idx_v, rows_v, sem):
        wid = lax.axis_index("s") * info.num_cores + lax.axis_index("c")
        base = wid * per_w

        def body(i, carry):
            off = base + i * CH
            pltpu.sync_copy(idx_hbm.at[pl.ds(off, CH)], idx_v)
            pltpu.async_copy(table_hbm.at[idx_v], rows_v, sem).wait()
            pltpu.sync_copy(rows_v, out_hbm.at[pl.ds(off, CH)])
            return carry

        lax.fori_loop(0, n_ch, body, 0)

    return k(table, flat_idx)


def _tc_ln_body(g_ref, seg_ref, pos_ref, segtab_ref, gam_ref, bet_ref, out_ref):
    emb = g_ref[...]
    pos = pos_ref[...]
    segf = seg_ref[...].astype(jnp.float32)[..., None]
    s0 = segtab_ref[0, :][None, None, :]
    s1 = segtab_ref[1, :][None, None, :]
    x = emb + pos[None, :, :] + s0 + segf * (s1 - s0)
    mean = jnp.mean(x, axis=-1, keepdims=True)
    xc = x - mean
    var = jnp.mean(xc * xc, axis=-1, keepdims=True)
    inv = lax.rsqrt(var + 1e-5)
    out_ref[...] = xc * inv * gam_ref[...] + bet_ref[...]


def _tc_ln(gathered, segments, pos_table, seg_table, gamma, beta):
    B, L, D = gathered.shape
    BB = 8
    return pl.pallas_call(
        _tc_ln_body,
        grid=(B // BB,),
        in_specs=[
            pl.BlockSpec((BB, L, D), lambda i: (i, 0, 0)),
            pl.BlockSpec((BB, L), lambda i: (i, 0)),
            pl.BlockSpec((L, D), lambda i: (0, 0)),
            pl.BlockSpec((seg_table.shape[0], D), lambda i: (0, 0)),
            pl.BlockSpec((D,), lambda i: (0,)),
            pl.BlockSpec((D,), lambda i: (0,)),
        ],
        out_specs=pl.BlockSpec((BB, L, D), lambda i: (i, 0, 0)),
        out_shape=jax.ShapeDtypeStruct((B, L, D), jnp.float32),
    )(gathered, segments, pos_table, seg_table, gamma, beta)


def kernel(x, segments, token_table, pos_table, seg_table, ln_gamma, ln_beta):
    B, L = x.shape
    V, D = token_table.shape
    flat_idx = x.reshape(B * L).astype(jnp.int32)
    gathered = _sc_gather(token_table, flat_idx)
    return _tc_ln(gathered.reshape(B, L, D), segments.astype(jnp.int32),
                  pos_table, seg_table, ln_gamma, ln_beta)


# speedup vs baseline: 6.5110x; 1.0550x over previous
import functools

import jax
import jax.numpy as jnp
from jax import lax
from jax.experimental import pallas as pl
from jax.experimental.pallas import tpu as pltpu
from jax.experimental.pallas import tpu_sc as plsc


def _sc_gather(table, flat_idx):
    V, D = table.shape
    N = flat_idx.shape[0]
    info = plsc.get_sparse_core_info()
    NW = info.num_cores * info.num_subcores
    per_w = N // NW
    CH = 256
    n_ch = per_w // CH
    NBUF = 3
    assert per_w % CH == 0 and N % NW == 0

    mesh = plsc.VectorSubcoreMesh(core_axis_name="c", subcore_axis_name="s")

    @functools.partial(
        pl.kernel,
        mesh=mesh,
        out_type=jax.ShapeDtypeStruct((N, D), jnp.float32),
        scratch_types=[
            pltpu.VMEM((per_w,), jnp.int32),
            pltpu.VMEM((NBUF, CH, D), jnp.float32),
            pltpu.SemaphoreType.DMA,
            pltpu.SemaphoreType.DMA((NBUF,)),
            pltpu.SemaphoreType.DMA((NBUF,)),
        ],
    )
    def k(table_hbm, idx_hbm, out_hbm, idx_v, rows_v, sem_i, sem_g, sem_o):
        wid = lax.axis_index("s") * info.num_cores + lax.axis_index("c")
        base = wid * per_w

        pltpu.async_copy(idx_hbm.at[pl.ds(base, per_w)], idx_v, sem_i).wait()

        def gather_start(g):
            b = g % NBUF
            return pltpu.async_copy(
                table_hbm.at[idx_v.at[pl.ds(g * CH, CH)]],
                rows_v.at[b], sem_g.at[b])

        def out_start(g):
            b = g % NBUF
            return pltpu.async_copy(
                rows_v.at[b], out_hbm.at[pl.ds(base + g * CH, CH)],
                sem_o.at[b])

        gathers = {}
        outs = {}
        gathers[0] = gather_start(0)
        if n_ch > 1:
            gathers[1] = gather_start(1)
        for g in range(n_ch):
            gathers.pop(g).wait()
            outs[g] = out_start(g)
            nxt = g + 2
            if nxt < n_ch:
                prev = nxt - NBUF
                if prev >= 0:
                    outs.pop(prev).wait()
                gathers[nxt] = gather_start(nxt)
        for g in sorted(outs):
            outs.pop(g).wait()

    return k(table, flat_idx)


def _tc_ln_body(g_ref, seg_ref, pos_ref, segtab_ref, gam_ref, bet_ref, out_ref):
    emb = g_ref[...]
    pos = pos_ref[...]
    segf = seg_ref[...].astype(jnp.float32)[..., None]
    s0 = segtab_ref[0, :][None, None, :]
    s1 = segtab_ref[1, :][None, None, :]
    x = emb + pos[None, :, :] + s0 + segf * (s1 - s0)
    mean = jnp.mean(x, axis=-1, keepdims=True)
    xc = x - mean
    var = jnp.mean(xc * xc, axis=-1, keepdims=True)
    inv = lax.rsqrt(var + 1e-5)
    out_ref[...] = xc * inv * gam_ref[...] + bet_ref[...]


def _tc_ln(gathered, segments, pos_table, seg_table, gamma, beta):
    B, L, D = gathered.shape
    BB = 8
    return pl.pallas_call(
        _tc_ln_body,
        grid=(B // BB,),
        in_specs=[
            pl.BlockSpec((BB, L, D), lambda i: (i, 0, 0)),
            pl.BlockSpec((BB, L), lambda i: (i, 0)),
            pl.BlockSpec((L, D), lambda i: (0, 0)),
            pl.BlockSpec((seg_table.shape[0], D), lambda i: (0, 0)),
            pl.BlockSpec((D,), lambda i: (0,)),
            pl.BlockSpec((D,), lambda i: (0,)),
        ],
        out_specs=pl.BlockSpec((BB, L, D), lambda i: (i, 0, 0)),
        out_shape=jax.ShapeDtypeStruct((B, L, D), jnp.float32),
    )(gathered, segments, pos_table, seg_table, gamma, beta)


def kernel(x, segments, token_table, pos_table, seg_table, ln_gamma, ln_beta):
    B, L = x.shape
    V, D = token_table.shape
    flat_idx = x.reshape(B * L).astype(jnp.int32)
    gathered = _sc_gather(token_table, flat_idx)
    return _tc_ln(gathered.reshape(B, L, D), segments.astype(jnp.int32),
                  pos_table, seg_table, ln_gamma, ln_beta)
